# baseline (speedup 1.0000x reference)
import functools

import jax
import jax.numpy as jnp
from jax import lax
from jax.experimental import pallas as pl
from jax.experimental.pallas import tpu as pltpu
from jax.experimental.pallas import tpu_sc as plsc

_NBUF = 4


def _build_gather(B, H, V, D, rows_per_w, R, num_cores):
    mesh = plsc.VectorSubcoreMesh(core_axis_name="c", subcore_axis_name="s")
    n_chunks = rows_per_w // R
    n_groups = n_chunks // _NBUF

    @functools.partial(
        pl.kernel,
        mesh=mesh,
        out_type=jax.ShapeDtypeStruct((B, H, D), jnp.float32),
        scratch_types=[
            [pltpu.VMEM((R, H), jnp.int32)] * _NBUF,
            pltpu.VMEM((_NBUF, R, H, D), jnp.float32),
            [pltpu.SemaphoreType.DMA] * _NBUF,
            [pltpu.SemaphoreType.DMA] * _NBUF,
        ],
        compiler_params=pltpu.CompilerParams(use_tc_tiling_on_sc=False),
    )
    def gather_kernel(x_hbm, table_hbm, out_hbm, idx_bufs, rows_v, gsems, osems):
        wid = lax.axis_index("s") * num_cores + lax.axis_index("c")
        base = wid * rows_per_w

        def start_gather(i, b):
            pltpu.sync_copy(x_hbm.at[pl.ds(base + i * R, R), :], idx_bufs[b])
            for r in range(R):
                pltpu.async_copy(
                    table_hbm.at[idx_bufs[b].at[r]], rows_v.at[b, r], gsems[b]
                )

        def wait_gather(b):
            for r in range(R):
                pltpu.make_async_copy(
                    table_hbm.at[idx_bufs[b].at[r]], rows_v.at[b, r], gsems[b]
                ).wait()

        for b in range(_NBUF):
            start_gather(b, b)

        def group_body(g, _):
            for b in range(_NBUF):
                i = g * _NBUF + b
                wait_gather(b)
                out_copy = pltpu.async_copy(
                    rows_v.at[b], out_hbm.at[pl.ds(base + i * R, R), :, :], osems[b]
                )
                out_copy.wait()

                @pl.when(g < n_groups - 1)
                def _():
                    start_gather(i + _NBUF, b)

            return _

        lax.fori_loop(0, n_groups, group_body, None)

    return gather_kernel


_NSLICE = 5


def kernel(x, weight):
    B, H = x.shape
    V, D = weight.shape

    info = plsc.get_sparse_core_info()
    NW = info.num_cores * info.num_subcores
    rows_per_w = B // NW
    Hs = H // _NSLICE
    R = 64

    gather = _build_gather(B, Hs, V, D, rows_per_w, R, info.num_cores)
    xi = x.astype(jnp.int32)
    outs = [
        gather(lax.slice_in_dim(xi, s * Hs, (s + 1) * Hs, axis=1), weight)
        for s in range(_NSLICE)
    ]
    return jnp.concatenate(outs, axis=1)


# speedup vs baseline: 1.0942x; 1.0942x over previous
import functools

import jax
import jax.numpy as jnp
from jax import lax
from jax.experimental import pallas as pl
from jax.experimental.pallas import tpu as pltpu
from jax.experimental.pallas import tpu_sc as plsc

_NBUF = 4


def _build_gather(B, H, V, D, rows_per_w, R, num_cores):
    mesh = plsc.VectorSubcoreMesh(core_axis_name="c", subcore_axis_name="s")
    n_chunks = rows_per_w // R
    n_groups = n_chunks // _NBUF

    @functools.partial(
        pl.kernel,
        mesh=mesh,
        out_type=jax.ShapeDtypeStruct((B, H, D), jnp.float32),
        scratch_types=[
            [pltpu.VMEM((R, H), jnp.int32)] * _NBUF,
            pltpu.VMEM((_NBUF, R, H, D), jnp.float32),
            [pltpu.SemaphoreType.DMA] * _NBUF,
            [pltpu.SemaphoreType.DMA] * _NBUF,
        ],
        compiler_params=pltpu.CompilerParams(use_tc_tiling_on_sc=False),
    )
    def gather_kernel(x_hbm, table_hbm, out_hbm, idx_bufs, rows_v, gsems, osems):
        wid = lax.axis_index("s") * num_cores + lax.axis_index("c")
        base = wid * rows_per_w

        def start_gather(i, b):
            pltpu.sync_copy(x_hbm.at[pl.ds(base + i * R, R), :], idx_bufs[b])
            for r in range(R):
                pltpu.async_copy(
                    table_hbm.at[idx_bufs[b].at[r]], rows_v.at[b, r], gsems[b]
                )

        def wait_gather(b):
            for r in range(R):
                pltpu.make_async_copy(
                    table_hbm.at[idx_bufs[b].at[r]], rows_v.at[b, r], gsems[b]
                ).wait()

        for b in range(_NBUF):
            start_gather(b, b)

        def group_body(g, _):
            for b in range(_NBUF):
                i = g * _NBUF + b
                wait_gather(b)
                out_copy = pltpu.async_copy(
                    rows_v.at[b], out_hbm.at[pl.ds(base + i * R, R), :, :], osems[b]
                )
                out_copy.wait()

                @pl.when(g < n_groups - 1)
                def _():
                    start_gather(i + _NBUF, b)

            return _

        lax.fori_loop(0, n_groups, group_body, None)

    return gather_kernel


def kernel(x, weight):
    B, H = x.shape
    V, D = weight.shape

    info = plsc.get_sparse_core_info()
    NW = info.num_cores * info.num_subcores
    rows_per_w = B // NW
    R = 16

    gather = _build_gather(B, H, V, D, rows_per_w, R, info.num_cores)
    return gather(x.astype(jnp.int32), weight)
